# Initial kernel scaffold; baseline (speedup 1.0000x reference)
import jax
import jax.numpy as jnp
from jax.experimental import pallas as pl


def kernel(x, edge_index, W1, b1, W2, b2, W3, b3):
    raise NotImplementedError("write your pallas kernel here")


# speedup vs baseline: 3.8303x; 3.8303x over previous
import functools

import jax
import jax.numpy as jnp
from jax import lax
from jax.experimental import pallas as pl
from jax.experimental.pallas import tpu as pltpu
from jax.experimental.pallas import tpu_sc as plsc

_GATHER_DNUMS = lax.GatherDimensionNumbers(
    offset_dims=(), collapsed_slice_dims=(0,), start_index_map=(0,))


def _lane_perm(v, idx):
    return lax.gather(v, idx[:, None], dimension_numbers=_GATHER_DNUMS,
                      slice_sizes=(1,),
                      mode=lax.GatherScatterMode.PROMISE_IN_BOUNDS)


def _node_precompute(x, W1, b1, W2, b2):
    N, D = x.shape
    H = W2.shape[1]
    BLK = 1000
    grid = N // BLK

    dot = functools.partial(jnp.dot, preferred_element_type=jnp.float32,
                            precision=lax.Precision.HIGHEST)

    def body(x_ref, w1_ref, b1_ref, w2_ref, b2_ref, a_ref, b_ref):
        xb = x_ref[...]
        w1 = w1_ref[...]
        w2 = w2_ref[...]
        ha = dot(xb, w1[:D])
        hb = dot(xb, w1[D:])
        c = dot(b1_ref[...], w2) + b2_ref[...]
        a_ref[...] = dot(ha, w2) + c
        b_ref[...] = dot(hb, w2)

    return pl.pallas_call(
        body,
        grid=(grid,),
        in_specs=[
            pl.BlockSpec((BLK, D), lambda i: (i, 0)),
            pl.BlockSpec((2 * D, H), lambda i: (0, 0)),
            pl.BlockSpec((1, H), lambda i: (0, 0)),
            pl.BlockSpec((H, H), lambda i: (0, 0)),
            pl.BlockSpec((1, H), lambda i: (0, 0)),
        ],
        out_specs=[
            pl.BlockSpec((BLK, H), lambda i: (i, 0)),
            pl.BlockSpec((BLK, H), lambda i: (i, 0)),
        ],
        out_shape=[
            jax.ShapeDtypeStruct((N, H), jnp.float32),
            jax.ShapeDtypeStruct((N, H), jnp.float32),
        ],
    )(x, W1, b1.reshape(1, H), W2, b2.reshape(1, H))


def _sc_edges(A, B, x, src, dst, w3):
    E = src.shape[0]
    N, D = x.shape
    H = A.shape[1]
    info = plsc.get_sparse_core_info()
    NC, NS = info.num_cores, info.num_subcores
    NW = NC * NS
    per_w = E // NW
    C = 400
    SUB = 80
    NSUB = C // SUB
    n_chunks = per_w // C
    mesh = plsc.VectorSubcoreMesh(core_axis_name="c", subcore_axis_name="s")

    @functools.partial(
        pl.kernel,
        mesh=mesh,
        compiler_params=pltpu.CompilerParams(use_tc_tiling_on_sc=False),
        out_type=[
            jax.ShapeDtypeStruct((E,), jnp.float32),
            jax.ShapeDtypeStruct((E, D), jnp.float32),
        ],
        scratch_types=[
            pltpu.VMEM((NSUB, SUB), jnp.int32),
            pltpu.VMEM((NSUB, SUB), jnp.int32),
            pltpu.VMEM((C, H), jnp.float32),
            pltpu.VMEM((C, H), jnp.float32),
            pltpu.VMEM((C, D), jnp.float32),
            pltpu.VMEM((H,), jnp.float32),
            pltpu.VMEM((C,), jnp.float32),
            pltpu.SemaphoreType.DMA,
            pltpu.SemaphoreType.DMA,
        ],
    )
    def k(a_hbm, b_hbm, x_hbm, src_hbm, dst_hbm, w3_hbm, ew_hbm, nd_hbm,
          src_v, dst_v, a_v, b_v, x_v, w3_v, ew_v, sem_x, sem_ab):
        wid = lax.axis_index("s") * NC + lax.axis_index("c")
        pltpu.sync_copy(w3_hbm, w3_v)
        w3s = [w3_v[pl.ds(16 * t, 16)] for t in range(H // 16)]
        lanes = lax.iota(jnp.int32, 16)
        perms = {kk: lanes ^ kk for kk in (1, 2, 4, 8)}
        masks = {kk: (lanes & kk) == 0 for kk in (1, 2, 4, 8)}
        bitrev = (((lanes & 1) << 3) | ((lanes & 2) << 1)
                  | ((lanes & 4) >> 1) | ((lanes & 8) >> 3))

        def chunk_body(i, carry):
            base = pl.multiple_of(wid * per_w + i * C, SUB)
            for s in range(NSUB):
                pltpu.sync_copy(src_hbm.at[pl.ds(base + s * SUB, SUB)],
                                src_v.at[s])
                pltpu.sync_copy(dst_hbm.at[pl.ds(base + s * SUB, SUB)],
                                dst_v.at[s])
            x_cps = []
            ab_cps = []
            for s in range(NSUB):
                x_cps.append(pltpu.async_copy(
                    x_hbm.at[src_v.at[s]], x_v.at[pl.ds(s * SUB, SUB)], sem_x))
                ab_cps.append(pltpu.async_copy(
                    a_hbm.at[dst_v.at[s]], a_v.at[pl.ds(s * SUB, SUB)], sem_ab))
                ab_cps.append(pltpu.async_copy(
                    b_hbm.at[src_v.at[s]], b_v.at[pl.ds(s * SUB, SUB)], sem_ab))
            for cp in ab_cps:
                cp.wait()

            def group_body(g, carry2):
                e0 = g * 16
                folds = []
                for kk in range(16):
                    e = e0 + kk
                    r = None
                    for j in range(H // 16):
                        h = a_v[e, pl.ds(16 * j, 16)] + b_v[e, pl.ds(16 * j, 16)]
                        p = jnp.maximum(h, 0.0) * w3s[j]
                        r = p if r is None else r + p
                    folds.append(r)
                cur = folds
                step = 8
                while len(cur) > 1:
                    mk, pm = masks[step], perms[step]
                    cur = [jnp.where(mk, a, _lane_perm(b, pm))
                           + jnp.where(mk, _lane_perm(a, pm), b)
                           for a, b in zip(cur[::2], cur[1::2])]
                    step //= 2
                ew_v[pl.ds(e0, 16)] = _lane_perm(cur[0], bitrev)
                return carry2

            lax.fori_loop(0, C // 16, group_body, 0)
            pltpu.sync_copy(ew_v, ew_hbm.at[pl.ds(base, C)])
            for cp in x_cps:
                cp.wait()
            pltpu.sync_copy(x_v, nd_hbm.at[pl.ds(base, C)])
            return carry

        lax.fori_loop(0, n_chunks, chunk_body, 0)

    return k(A, B, x, src, dst, w3)


def kernel(x, edge_index, W1, b1, W2, b2, W3, b3):
    src = edge_index[0]
    dst = edge_index[1]
    A, B = _node_precompute(x, W1, b1, W2, b2)
    ew, node_data = _sc_edges(A, B, x, src, dst, W3.reshape(-1))
    edge_weight = ew.reshape(-1, 1) + b3
    return edge_weight, node_data


# speedup vs baseline: 5.3408x; 1.3943x over previous
import functools

import jax
import jax.numpy as jnp
from jax import lax
from jax.experimental import pallas as pl
from jax.experimental.pallas import tpu as pltpu
from jax.experimental.pallas import tpu_sc as plsc

_GATHER_DNUMS = lax.GatherDimensionNumbers(
    offset_dims=(), collapsed_slice_dims=(0,), start_index_map=(0,))


def _lane_perm(v, idx):
    return lax.gather(v, idx[:, None], dimension_numbers=_GATHER_DNUMS,
                      slice_sizes=(1,),
                      mode=lax.GatherScatterMode.PROMISE_IN_BOUNDS)


def _node_precompute(x, W1, b1, W2, b2):
    N, D = x.shape
    H = W2.shape[1]
    BLK = 1000
    grid = N // BLK

    dot = functools.partial(jnp.dot, preferred_element_type=jnp.float32,
                            precision=lax.Precision.HIGHEST)

    def body(x_ref, w1_ref, b1_ref, w2_ref, b2_ref, a_ref, b_ref):
        xb = x_ref[...]
        w1 = w1_ref[...]
        w2 = w2_ref[...]
        ha = dot(xb, w1[:D])
        hb = dot(xb, w1[D:])
        c = dot(b1_ref[...], w2) + b2_ref[...]
        a_ref[...] = dot(ha, w2) + c
        b_ref[...] = dot(hb, w2)

    return pl.pallas_call(
        body,
        grid=(grid,),
        in_specs=[
            pl.BlockSpec((BLK, D), lambda i: (i, 0)),
            pl.BlockSpec((2 * D, H), lambda i: (0, 0)),
            pl.BlockSpec((1, H), lambda i: (0, 0)),
            pl.BlockSpec((H, H), lambda i: (0, 0)),
            pl.BlockSpec((1, H), lambda i: (0, 0)),
        ],
        out_specs=[
            pl.BlockSpec((BLK, H), lambda i: (i, 0)),
            pl.BlockSpec((BLK, H), lambda i: (i, 0)),
        ],
        out_shape=[
            jax.ShapeDtypeStruct((N, H), jnp.float32),
            jax.ShapeDtypeStruct((N, H), jnp.float32),
        ],
    )(x, W1, b1.reshape(1, H), W2, b2.reshape(1, H))


def _sc_edges(A, B, x, src2d, dst2d, w3b3):
    E = src2d.shape[0] * src2d.shape[1]
    N, D = x.shape
    H = A.shape[1]
    info = plsc.get_sparse_core_info()
    NC, NS = info.num_cores, info.num_subcores
    NW = NC * NS
    per_w = E // NW
    SUB = src2d.shape[1]
    C = 4 * SUB
    NSUB = C // SUB
    n_chunks = per_w // C
    rows_w = per_w // SUB
    mesh = plsc.VectorSubcoreMesh(core_axis_name="c", subcore_axis_name="s")

    @functools.partial(
        pl.kernel,
        mesh=mesh,
        compiler_params=pltpu.CompilerParams(use_tc_tiling_on_sc=False),
        out_type=[
            jax.ShapeDtypeStruct((E,), jnp.float32),
            jax.ShapeDtypeStruct((E, D), jnp.float32),
        ],
        scratch_types=[
            pltpu.VMEM((rows_w, SUB), jnp.int32),
            pltpu.VMEM((rows_w, SUB), jnp.int32),
            pltpu.VMEM((C, H), jnp.float32),
            pltpu.VMEM((C, H), jnp.float32),
            pltpu.VMEM((C, D), jnp.float32),
            pltpu.VMEM((2 * H,), jnp.float32),
            pltpu.VMEM((C,), jnp.float32),
            pltpu.SemaphoreType.DMA,
            pltpu.SemaphoreType.DMA,
        ],
    )
    def k(a_hbm, b_hbm, x_hbm, src_hbm, dst_hbm, w3_hbm, ew_hbm, nd_hbm,
          src_v, dst_v, a_v, b_v, x_v, w3_v, ew_v, sem_x, sem_ab):
        wid = lax.axis_index("s") * NC + lax.axis_index("c")
        pltpu.sync_copy(w3_hbm, w3_v)
        row0 = wid * rows_w
        pltpu.sync_copy(src_hbm.at[pl.ds(row0, rows_w)], src_v)
        pltpu.sync_copy(dst_hbm.at[pl.ds(row0, rows_w)], dst_v)
        w3s = [w3_v[pl.ds(16 * t, 16)] for t in range(H // 16)]
        b3v = w3_v[pl.ds(H, 16)]
        lanes = lax.iota(jnp.int32, 16)
        perms = {kk: lanes ^ kk for kk in (1, 2, 4, 8)}
        masks = {kk: (lanes & kk) == 0 for kk in (1, 2, 4, 8)}
        bitrev = (((lanes & 1) << 3) | ((lanes & 2) << 1)
                  | ((lanes & 4) >> 1) | ((lanes & 8) >> 3))

        def chunk_body(i, carry):
            base = pl.multiple_of(wid * per_w + i * C, SUB)
            x_cps = []
            ab_cps = []
            for s in range(NSUB):
                r = i * NSUB + s
                x_cps.append(pltpu.async_copy(
                    x_hbm.at[src_v.at[r]], x_v.at[pl.ds(s * SUB, SUB)], sem_x))
                ab_cps.append(pltpu.async_copy(
                    a_hbm.at[dst_v.at[r]], a_v.at[pl.ds(s * SUB, SUB)], sem_ab))
                ab_cps.append(pltpu.async_copy(
                    b_hbm.at[src_v.at[r]], b_v.at[pl.ds(s * SUB, SUB)], sem_ab))
            for cp in ab_cps:
                cp.wait()

            def group_body(g, carry2):
                e0 = g * 16
                folds = []
                for kk in range(16):
                    e = e0 + kk
                    r = None
                    for j in range(H // 16):
                        h = a_v[e, pl.ds(16 * j, 16)] + b_v[e, pl.ds(16 * j, 16)]
                        p = jnp.maximum(h, 0.0) * w3s[j]
                        r = p if r is None else r + p
                    folds.append(r)
                cur = folds
                step = 8
                while len(cur) > 1:
                    mk, pm = masks[step], perms[step]
                    cur = [jnp.where(mk, a, _lane_perm(b, pm))
                           + jnp.where(mk, _lane_perm(a, pm), b)
                           for a, b in zip(cur[::2], cur[1::2])]
                    step //= 2
                ew_v[pl.ds(e0, 16)] = _lane_perm(cur[0], bitrev) + b3v[0]
                return carry2

            lax.fori_loop(0, C // 16, group_body, 0)
            pltpu.sync_copy(ew_v, ew_hbm.at[pl.ds(base, C)])
            for cp in x_cps:
                cp.wait()
            pltpu.sync_copy(x_v, nd_hbm.at[pl.ds(base, C)])
            return carry

        lax.fori_loop(0, n_chunks, chunk_body, 0)

    return k(A, B, x, src2d, dst2d, w3b3)


def kernel(x, edge_index, W1, b1, W2, b2, W3, b3):
    SUB = 100
    E = edge_index.shape[1]
    H = W2.shape[1]
    src2d = edge_index[0].reshape(E // SUB, SUB)
    dst2d = edge_index[1].reshape(E // SUB, SUB)
    w3b3 = jnp.concatenate(
        [W3.reshape(-1), b3.reshape(-1),
         jnp.zeros((2 * H - H - b3.size,), jnp.float32)])
    A, B = _node_precompute(x, W1, b1, W2, b2)
    ew, node_data = _sc_edges(A, B, x, src2d, dst2d, w3b3)
    return ew.reshape(-1, 1), node_data


# speedup vs baseline: 6.8491x; 1.2824x over previous
import functools

import jax
import jax.numpy as jnp
from jax import lax
from jax.experimental import pallas as pl
from jax.experimental.pallas import tpu as pltpu
from jax.experimental.pallas import tpu_sc as plsc

_GATHER_DNUMS = lax.GatherDimensionNumbers(
    offset_dims=(), collapsed_slice_dims=(0,), start_index_map=(0,))


def _lane_perm(v, idx):
    return lax.gather(v, idx[:, None], dimension_numbers=_GATHER_DNUMS,
                      slice_sizes=(1,),
                      mode=lax.GatherScatterMode.PROMISE_IN_BOUNDS)


def _node_precompute(x, W1, b1, W2, b2):
    N, D = x.shape
    H = W2.shape[1]
    BLK = 1000
    grid = N // BLK

    dot = functools.partial(jnp.dot, preferred_element_type=jnp.float32,
                            precision=lax.Precision.HIGHEST)

    def body(x_ref, w1_ref, b1_ref, w2_ref, b2_ref, a_ref, b_ref):
        xb = x_ref[...]
        w1 = w1_ref[...]
        w2 = w2_ref[...]
        ha = dot(xb, w1[:D])
        hb = dot(xb, w1[D:])
        c = dot(b1_ref[...], w2) + b2_ref[...]
        a_ref[...] = dot(ha, w2) + c
        b_ref[...] = dot(hb, w2)

    return pl.pallas_call(
        body,
        grid=(grid,),
        in_specs=[
            pl.BlockSpec((BLK, D), lambda i: (i, 0)),
            pl.BlockSpec((2 * D, H), lambda i: (0, 0)),
            pl.BlockSpec((1, H), lambda i: (0, 0)),
            pl.BlockSpec((H, H), lambda i: (0, 0)),
            pl.BlockSpec((1, H), lambda i: (0, 0)),
        ],
        out_specs=[
            pl.BlockSpec((BLK, H), lambda i: (i, 0)),
            pl.BlockSpec((BLK, H), lambda i: (i, 0)),
        ],
        out_shape=[
            jax.ShapeDtypeStruct((N, H), jnp.float32),
            jax.ShapeDtypeStruct((N, H), jnp.float32),
        ],
    )(x, W1, b1.reshape(1, H), W2, b2.reshape(1, H))


def _sc_edges(A, B, x, src1d, dst1d, w3b3):
    E = src1d.shape[0]
    N, D = x.shape
    H = A.shape[1]
    info = plsc.get_sparse_core_info()
    NC, NS = info.num_cores, info.num_subcores
    NW = NC * NS
    per_w = E // NW
    C = 80
    n_chunks = per_w // C
    mesh = plsc.VectorSubcoreMesh(core_axis_name="c", subcore_axis_name="s")

    @functools.partial(
        pl.kernel,
        mesh=mesh,
        compiler_params=pltpu.CompilerParams(use_tc_tiling_on_sc=False),
        out_type=[
            jax.ShapeDtypeStruct((E,), jnp.float32),
            jax.ShapeDtypeStruct((E, D), jnp.float32),
        ],
        scratch_types=[
            pltpu.VMEM((per_w,), jnp.int32),
            pltpu.VMEM((per_w,), jnp.int32),
            pltpu.VMEM((C, H), jnp.float32),
            pltpu.VMEM((C, H), jnp.float32),
            pltpu.VMEM((C, H), jnp.float32),
            pltpu.VMEM((C, H), jnp.float32),
            pltpu.VMEM((C, H), jnp.float32),
            pltpu.VMEM((C, H), jnp.float32),
            pltpu.VMEM((C, D), jnp.float32),
            pltpu.VMEM((C, D), jnp.float32),
            pltpu.VMEM((C, D), jnp.float32),
            pltpu.VMEM((2 * H,), jnp.float32),
            pltpu.VMEM((C,), jnp.float32),
            pltpu.VMEM((C,), jnp.float32),
            pltpu.VMEM((C,), jnp.float32),
            pltpu.SemaphoreType.DMA,
            pltpu.SemaphoreType.DMA,
            pltpu.SemaphoreType.DMA,
        ],
    )
    def k(a_hbm, b_hbm, x_hbm, src_hbm, dst_hbm, w3_hbm, ew_hbm, nd_hbm,
          src_v, dst_v, a0, a1, a2, b0, b1, b2, x0, x1, x2, w3_v,
          ew0, ew1, ew2, sem_ab, sem_x, sem_w):
        wid = lax.axis_index("s") * NC + lax.axis_index("c")
        a_s, b_s = (a0, a1, a2), (b0, b1, b2)
        x_s, ew_s = (x0, x1, x2), (ew0, ew1, ew2)
        pltpu.sync_copy(w3_hbm, w3_v)
        e0w = wid * per_w
        pltpu.sync_copy(src_hbm.at[pl.ds(e0w, per_w)], src_v)
        pltpu.sync_copy(dst_hbm.at[pl.ds(e0w, per_w)], dst_v)
        w3s = [w3_v[pl.ds(16 * t, 16)] for t in range(H // 16)]
        b3v = w3_v[pl.ds(H, 16)]
        lanes = lax.iota(jnp.int32, 16)
        perms = {kk: lanes ^ kk for kk in (1, 2, 4, 8)}
        masks = {kk: (lanes & kk) == 0 for kk in (1, 2, 4, 8)}
        bitrev = (((lanes & 1) << 3) | ((lanes & 2) << 1)
                  | ((lanes & 4) >> 1) | ((lanes & 8) >> 3))

        def fire(i, s):
            idx = pl.ds(i * C, C)
            pltpu.async_copy(a_hbm.at[dst_v.at[idx]], a_s[s], sem_ab)
            pltpu.async_copy(b_hbm.at[src_v.at[idx]], b_s[s], sem_ab)
            pltpu.async_copy(x_hbm.at[src_v.at[idx]], x_s[s], sem_x)

        def wait_ab(s):
            pltpu.make_async_copy(a_hbm.at[pl.ds(0, C)], a_s[s], sem_ab).wait()
            pltpu.make_async_copy(b_hbm.at[pl.ds(0, C)], b_s[s], sem_ab).wait()

        def wait_x(s):
            pltpu.make_async_copy(x_hbm.at[pl.ds(0, C)], x_s[s], sem_x).wait()

        def put(i, s):
            base = pl.multiple_of(e0w + i * C, C)
            pltpu.async_copy(ew_s[s], ew_hbm.at[pl.ds(base, C)], sem_w)
            pltpu.async_copy(x_s[s], nd_hbm.at[pl.ds(base, C)], sem_w)

        def wait_put(s):
            pltpu.make_async_copy(ew_s[s], ew_hbm.at[pl.ds(0, C)], sem_w).wait()
            pltpu.make_async_copy(x_s[s], nd_hbm.at[pl.ds(0, C)], sem_w).wait()

        def compute(s):
            a_v, b_v, ew_v = a_s[s], b_s[s], ew_s[s]

            def group_body(g, carry2):
                e0 = g * 16
                folds = []
                for kk in range(16):
                    e = e0 + kk
                    r = None
                    for j in range(H // 16):
                        h = a_v[e, pl.ds(16 * j, 16)] + b_v[e, pl.ds(16 * j, 16)]
                        p = jnp.maximum(h, 0.0) * w3s[j]
                        r = p if r is None else r + p
                    folds.append(r)
                cur = folds
                step = 8
                while len(cur) > 1:
                    mk, pm = masks[step], perms[step]
                    cur = [jnp.where(mk, a, _lane_perm(b, pm))
                           + jnp.where(mk, _lane_perm(a, pm), b)
                           for a, b in zip(cur[::2], cur[1::2])]
                    step //= 2
                ew_v[pl.ds(e0, 16)] = _lane_perm(cur[0], bitrev) + b3v[0]
                return carry2

            lax.fori_loop(0, C // 16, group_body, 0)

        fire(0, 0)
        fire(1, 1)
        wait_ab(0)
        compute(0)
        wait_x(0)
        fire(2, 2)
        put(0, 0)
        wait_ab(1)
        compute(1)
        wait_x(1)
        wait_put(0)
        fire(3, 0)
        put(1, 1)

        def steady(i, s):
            wait_ab(s)
            compute(s)
            wait_x(s)
            wait_put((s + 2) % 3)
            fire(i + 2, (s + 2) % 3)
            put(i, s)

        def tri_body(o, carry):
            i = 3 * o
            steady(i + 2, 2)
            steady(i + 3, 0)
            steady(i + 4, 1)
            return carry

        n_steady = n_chunks - 4
        lax.fori_loop(0, n_steady // 3, tri_body, 0)
        for i in range(n_chunks - 2 - (n_steady % 3), n_chunks - 2):
            steady(i, i % 3)
        s2 = (n_chunks - 2) % 3
        wait_ab(s2)
        compute(s2)
        wait_x(s2)
        wait_put((s2 + 2) % 3)
        put(n_chunks - 2, s2)
        s1 = (n_chunks - 1) % 3
        wait_ab(s1)
        compute(s1)
        wait_x(s1)
        wait_put((s1 + 2) % 3)
        put(n_chunks - 1, s1)
        wait_put(s1)

    return k(A, B, x, src1d, dst1d, w3b3)


def kernel(x, edge_index, W1, b1, W2, b2, W3, b3):
    H = W2.shape[1]
    w3b3 = jnp.concatenate(
        [W3.reshape(-1), b3.reshape(-1),
         jnp.zeros((2 * H - H - b3.size,), jnp.float32)])
    A, B = _node_precompute(x, W1, b1, W2, b2)
    ew, node_data = _sc_edges(A, B, x, edge_index[0], edge_index[1], w3b3)
    return ew.reshape(-1, 1), node_data


# speedup vs baseline: 7.0788x; 1.0335x over previous
import functools

import jax
import jax.numpy as jnp
from jax import lax
from jax.experimental import pallas as pl
from jax.experimental.pallas import tpu as pltpu
from jax.experimental.pallas import tpu_sc as plsc

_GATHER_DNUMS = lax.GatherDimensionNumbers(
    offset_dims=(), collapsed_slice_dims=(0,), start_index_map=(0,))


def _lane_perm(v, idx):
    return lax.gather(v, idx[:, None], dimension_numbers=_GATHER_DNUMS,
                      slice_sizes=(1,),
                      mode=lax.GatherScatterMode.PROMISE_IN_BOUNDS)


def _node_precompute(x, W1, b1, W2, b2):
    N, D = x.shape
    H = W2.shape[1]
    BLK = 1000
    grid = N // BLK

    dot = functools.partial(jnp.dot, preferred_element_type=jnp.float32,
                            precision=lax.Precision.HIGHEST)

    def body(x_ref, w1_ref, b1_ref, w2_ref, b2_ref, a_ref, b_ref):
        xb = x_ref[...]
        w1 = w1_ref[...]
        w2 = w2_ref[...]
        ha = dot(xb, w1[:D])
        hb = dot(xb, w1[D:])
        c = dot(b1_ref[...], w2) + b2_ref[...]
        a_ref[...] = dot(ha, w2) + c
        b_ref[...] = dot(hb, w2)

    return pl.pallas_call(
        body,
        grid=(grid,),
        in_specs=[
            pl.BlockSpec((BLK, D), lambda i: (i, 0)),
            pl.BlockSpec((2 * D, H), lambda i: (0, 0)),
            pl.BlockSpec((1, H), lambda i: (0, 0)),
            pl.BlockSpec((H, H), lambda i: (0, 0)),
            pl.BlockSpec((1, H), lambda i: (0, 0)),
        ],
        out_specs=[
            pl.BlockSpec((BLK, H), lambda i: (i, 0)),
            pl.BlockSpec((BLK, H), lambda i: (i, 0)),
        ],
        out_shape=[
            jax.ShapeDtypeStruct((N, H), jnp.float32),
            jax.ShapeDtypeStruct((N, H), jnp.float32),
        ],
    )(x, W1, b1.reshape(1, H), W2, b2.reshape(1, H))


def _sc_edges(A, B, x, edge_index, w3b3):
    E = edge_index.shape[1]
    N, D = x.shape
    H = A.shape[1]
    info = plsc.get_sparse_core_info()
    NC, NS = info.num_cores, info.num_subcores
    NW = NC * NS
    per_w = E // NW
    C = 80
    n_chunks = per_w // C
    mesh = plsc.VectorSubcoreMesh(core_axis_name="c", subcore_axis_name="s")

    @functools.partial(
        pl.kernel,
        mesh=mesh,
        compiler_params=pltpu.CompilerParams(use_tc_tiling_on_sc=False),
        out_type=[
            jax.ShapeDtypeStruct((E,), jnp.float32),
            jax.ShapeDtypeStruct((E, D), jnp.float32),
        ],
        scratch_types=[
            pltpu.VMEM((per_w,), jnp.int32),
            pltpu.VMEM((per_w,), jnp.int32),
            pltpu.VMEM((C, H), jnp.float32),
            pltpu.VMEM((C, H), jnp.float32),
            pltpu.VMEM((C, H), jnp.float32),
            pltpu.VMEM((C, H), jnp.float32),
            pltpu.VMEM((C, H), jnp.float32),
            pltpu.VMEM((C, H), jnp.float32),
            pltpu.VMEM((C, D), jnp.float32),
            pltpu.VMEM((C, D), jnp.float32),
            pltpu.VMEM((C, D), jnp.float32),
            pltpu.VMEM((2 * H,), jnp.float32),
            pltpu.VMEM((per_w,), jnp.float32),
            pltpu.SemaphoreType.DMA,
            pltpu.SemaphoreType.DMA,
            pltpu.SemaphoreType.DMA,
        ],
    )
    def k(a_hbm, b_hbm, x_hbm, ei_hbm, w3_hbm, ew_hbm, nd_hbm,
          src_v, dst_v, a0, a1, a2, b0, b1, b2, x0, x1, x2, w3_v,
          ew_v, sem_ab, sem_x, sem_w):
        wid = lax.axis_index("s") * NC + lax.axis_index("c")
        a_s, b_s = (a0, a1, a2), (b0, b1, b2)
        x_s = (x0, x1, x2)
        pltpu.sync_copy(w3_hbm, w3_v)
        e0w = wid * per_w
        pltpu.sync_copy(ei_hbm.at[0, pl.ds(e0w, per_w)], src_v)
        pltpu.sync_copy(ei_hbm.at[1, pl.ds(e0w, per_w)], dst_v)
        w3s = [w3_v[pl.ds(16 * t, 16)] for t in range(H // 16)]
        b3v = w3_v[pl.ds(H, 16)]
        lanes = lax.iota(jnp.int32, 16)
        perms = {kk: lanes ^ kk for kk in (1, 2, 4, 8)}
        masks = {kk: (lanes & kk) == 0 for kk in (1, 2, 4, 8)}
        bitrev = (((lanes & 1) << 3) | ((lanes & 2) << 1)
                  | ((lanes & 4) >> 1) | ((lanes & 8) >> 3))

        def fire(i, s):
            idx = pl.ds(i * C, C)
            pltpu.async_copy(a_hbm.at[dst_v.at[idx]], a_s[s], sem_ab)
            pltpu.async_copy(b_hbm.at[src_v.at[idx]], b_s[s], sem_ab)
            pltpu.async_copy(x_hbm.at[src_v.at[idx]], x_s[s], sem_x)

        def wait_ab(s):
            pltpu.make_async_copy(a_hbm.at[pl.ds(0, C)], a_s[s], sem_ab).wait()
            pltpu.make_async_copy(b_hbm.at[pl.ds(0, C)], b_s[s], sem_ab).wait()

        def wait_x(s):
            pltpu.make_async_copy(x_hbm.at[pl.ds(0, C)], x_s[s], sem_x).wait()

        def put(i, s):
            base = pl.multiple_of(e0w + i * C, C)
            pltpu.async_copy(x_s[s], nd_hbm.at[pl.ds(base, C)], sem_w)

        def wait_put(s):
            pltpu.make_async_copy(x_s[s], nd_hbm.at[pl.ds(0, C)], sem_w).wait()

        def compute(i, s):
            a_v, b_v = a_s[s], b_s[s]
            o0 = i * C

            def group_body(g, carry2):
                e0 = g * 16
                folds = []
                for kk in range(16):
                    e = e0 + kk
                    r = None
                    for j in range(H // 16):
                        h = a_v[e, pl.ds(16 * j, 16)] + b_v[e, pl.ds(16 * j, 16)]
                        p = jnp.maximum(h, 0.0) * w3s[j]
                        r = p if r is None else r + p
                    folds.append(r)
                cur = folds
                step = 8
                while len(cur) > 1:
                    mk, pm = masks[step], perms[step]
                    cur = [jnp.where(mk, a, _lane_perm(b, pm))
                           + jnp.where(mk, _lane_perm(a, pm), b)
                           for a, b in zip(cur[::2], cur[1::2])]
                    step //= 2
                ew_v[pl.ds(o0 + e0, 16)] = _lane_perm(cur[0], bitrev) + b3v[0]
                return carry2

            lax.fori_loop(0, C // 16, group_body, 0)

        fire(0, 0)
        fire(1, 1)
        wait_ab(0)
        compute(0, 0)
        wait_x(0)
        fire(2, 2)
        put(0, 0)
        wait_ab(1)
        compute(1, 1)
        wait_x(1)
        wait_put(0)
        fire(3, 0)
        put(1, 1)

        def steady(i, s):
            wait_ab(s)
            compute(i, s)
            wait_x(s)
            wait_put((s + 2) % 3)
            fire(i + 2, (s + 2) % 3)
            put(i, s)

        def tri_body(o, carry):
            i = 3 * o
            steady(i + 2, 2)
            steady(i + 3, 0)
            steady(i + 4, 1)
            return carry

        n_steady = n_chunks - 4
        lax.fori_loop(0, n_steady // 3, tri_body, 0)
        for i in range(n_chunks - 2 - (n_steady % 3), n_chunks - 2):
            steady(i, i % 3)
        s2 = (n_chunks - 2) % 3
        wait_ab(s2)
        compute(n_chunks - 2, s2)
        wait_x(s2)
        wait_put((s2 + 2) % 3)
        put(n_chunks - 2, s2)
        s1 = (n_chunks - 1) % 3
        wait_ab(s1)
        compute(n_chunks - 1, s1)
        wait_x(s1)
        wait_put((s1 + 2) % 3)
        put(n_chunks - 1, s1)
        pltpu.sync_copy(ew_v, ew_hbm.at[pl.ds(e0w, per_w)])
        wait_put(s1)

    return k(A, B, x, edge_index, w3b3)


def kernel(x, edge_index, W1, b1, W2, b2, W3, b3):
    H = W2.shape[1]
    w3b3 = jnp.concatenate(
        [W3.reshape(-1), b3.reshape(-1),
         jnp.zeros((2 * H - H - b3.size,), jnp.float32)])
    A, B = _node_precompute(x, W1, b1, W2, b2)
    ew, node_data = _sc_edges(A, B, x, edge_index, w3b3)
    return ew.reshape(-1, 1), node_data
